# Initial kernel scaffold; baseline (speedup 1.0000x reference)
import functools

import jax
import jax.numpy as jnp
from jax import lax
from jax.experimental import pallas as pl
from jax.experimental.pallas import tpu as pltpu
from jax.experimental.pallas import tpu_sc as plsc

N_NODES = 10000
N_EDGES = 320000
D_FEAT = 128
NUM_LAYERS = 3

NS = 16
B = 128
NB = 157
E_PAD = NS * NB * B
ROWS_PER_TILE = N_NODES // NS
ZCHUNK = 125


def _lightgcn_sc(x, edge_row, edge_col, edge_vals):
    mesh = plsc.VectorSubcoreMesh(
        core_axis_name="c", subcore_axis_name="s", num_cores=1)

    @functools.partial(
        pl.kernel,
        out_type=[
            jax.ShapeDtypeStruct((N_NODES, D_FEAT), jnp.float32),
            jax.ShapeDtypeStruct((N_NODES, D_FEAT), jnp.float32),
        ],
        mesh=mesh,
        scratch_types=[
            pltpu.VMEM_SHARED((N_NODES, D_FEAT), jnp.float32),
            pltpu.VMEM((B,), jnp.int32),
            pltpu.VMEM((B,), jnp.int32),
            pltpu.VMEM((B,), jnp.float32),
            pltpu.VMEM((B, D_FEAT), jnp.float32),
            pltpu.VMEM((ZCHUNK, D_FEAT), jnp.float32),
            pltpu.SemaphoreType.DMA,
        ],
    )
    def k(x_hbm, rows_hbm, cols_hbm, vals_hbm, out_hbm, h_hbm,
          accum, colv, rowv, valv, rbuf, zbuf, sem):
        sid = lax.axis_index("s")
        zero16 = jnp.zeros((16,), jnp.float32)

        def zrow(i, _):
            for kk in range(D_FEAT // 16):
                zbuf[i, pl.ds(kk * 16, 16)] = zero16
            return 0
        lax.fori_loop(0, ZCHUNK, zrow, 0)
        for j in range(ROWS_PER_TILE // ZCHUNK):
            pltpu.sync_copy(
                zbuf, accum.at[pl.ds(sid * ROWS_PER_TILE + j * ZCHUNK, ZCHUNK)])
        plsc.subcore_barrier()

        for layer in range(NUM_LAYERS):
            src = x_hbm if layer == 0 else h_hbm

            def batch_body(b, _):
                base = (sid * NB + b) * B
                pltpu.sync_copy(cols_hbm.at[pl.ds(base, B)], colv)
                pltpu.sync_copy(rows_hbm.at[pl.ds(base, B)], rowv)
                pltpu.sync_copy(vals_hbm.at[pl.ds(base, B)], valv)
                pltpu.async_copy(src.at[colv], rbuf, sem).wait()

                def row_body(i, _):
                    v = valv[i]
                    for kk in range(D_FEAT // 16):
                        sl = pl.ds(kk * 16, 16)
                        rbuf[i, sl] = rbuf[i, sl] * v
                    return 0
                lax.fori_loop(0, B, row_body, 0)
                pltpu.sync_copy(rbuf, accum.at[rowv], add=True)
                return 0

            lax.fori_loop(0, NB, batch_body, 0)
            plsc.subcore_barrier()

            dst = out_hbm if layer == NUM_LAYERS - 1 else h_hbm
            for j in range(ROWS_PER_TILE // ZCHUNK):
                r0 = sid * ROWS_PER_TILE + j * ZCHUNK
                pltpu.sync_copy(accum.at[pl.ds(r0, ZCHUNK)],
                                dst.at[pl.ds(r0, ZCHUNK)])
            if layer < NUM_LAYERS - 1:
                for j in range(ROWS_PER_TILE // ZCHUNK):
                    r0 = sid * ROWS_PER_TILE + j * ZCHUNK
                    pltpu.sync_copy(zbuf, accum.at[pl.ds(r0, ZCHUNK)])
                plsc.subcore_barrier()

    return k(x, edge_row, edge_col, edge_vals)


def kernel(x, edge_row, edge_col, edge_vals):
    pad = E_PAD - N_EDGES
    edge_row = jnp.concatenate([edge_row, jnp.zeros((pad,), jnp.int32)])
    edge_col = jnp.concatenate([edge_col, jnp.zeros((pad,), jnp.int32)])
    edge_vals = jnp.concatenate([edge_vals, jnp.zeros((pad,), jnp.float32)])
    out, _ = _lightgcn_sc(x, edge_row, edge_col, edge_vals)
    return out


# speedup vs baseline: 2.5018x; 2.5018x over previous
import functools

import jax
import jax.numpy as jnp
from jax import lax
from jax.experimental import pallas as pl
from jax.experimental.pallas import tpu as pltpu
from jax.experimental.pallas import tpu_sc as plsc

N_NODES = 10000
N_EDGES = 320000
D_FEAT = 128
NUM_LAYERS = 3

NS = 16
B = 128
NB = 157
E_PAD = NS * NB * B
ROWS_PER_TILE = 624
ZCHUNK = 208
TAIL_ROWS = N_NODES - NS * ROWS_PER_TILE


def _lightgcn_sc(x, edge_row, edge_col, edge_vals):
    mesh = plsc.VectorSubcoreMesh(
        core_axis_name="c", subcore_axis_name="s", num_cores=1)

    @functools.partial(
        pl.kernel,
        out_type=[
            jax.ShapeDtypeStruct((N_NODES, D_FEAT), jnp.float32),
            jax.ShapeDtypeStruct((N_NODES, D_FEAT), jnp.float32),
        ],
        mesh=mesh,
        scratch_types=[
            pltpu.VMEM_SHARED((N_NODES, D_FEAT), jnp.float32),
            pltpu.VMEM((B,), jnp.int32),
            pltpu.VMEM((B,), jnp.int32),
            pltpu.VMEM((B,), jnp.float32),
            pltpu.VMEM((B, D_FEAT), jnp.float32),
            pltpu.VMEM((ZCHUNK, D_FEAT), jnp.float32),
            pltpu.SemaphoreType.DMA,
        ],
    )
    def k(x_hbm, rows_hbm, cols_hbm, vals_hbm, out_hbm, h_hbm,
          accum, colv, rowv, valv, rbuf, zbuf, sem):
        sid = lax.axis_index("s")
        zero16 = jnp.zeros((16,), jnp.float32)

        def my_chunks():
            chunks = [
                (pl.multiple_of(sid * ROWS_PER_TILE + j * ZCHUNK, 8), ZCHUNK)
                for j in range(ROWS_PER_TILE // ZCHUNK)
            ]
            return chunks

        def zero_my_slice():
            for r0, sz in my_chunks():
                pltpu.sync_copy(zbuf.at[pl.ds(0, sz)], accum.at[pl.ds(r0, sz)])

            @pl.when(sid == NS - 1)
            def _():
                pltpu.sync_copy(zbuf.at[pl.ds(0, TAIL_ROWS)],
                                accum.at[pl.ds(NS * ROWS_PER_TILE, TAIL_ROWS)])

        def flush_my_slice(dst_ref):
            for r0, sz in my_chunks():
                pltpu.sync_copy(accum.at[pl.ds(r0, sz)],
                                dst_ref.at[pl.ds(r0, sz)])

            @pl.when(sid == NS - 1)
            def _():
                r0 = NS * ROWS_PER_TILE
                pltpu.sync_copy(accum.at[pl.ds(r0, TAIL_ROWS)],
                                dst_ref.at[pl.ds(r0, TAIL_ROWS)])

        def zrow(i, _):
            for kk in range(D_FEAT // 16):
                zbuf[i, pl.ds(kk * 16, 16)] = zero16
            return 0
        lax.fori_loop(0, ZCHUNK, zrow, 0)
        zero_my_slice()
        plsc.subcore_barrier()

        for layer in range(NUM_LAYERS):
            src = x_hbm if layer == 0 else h_hbm

            def batch_body(b, _):
                base = (sid * NB + b) * B
                pltpu.sync_copy(cols_hbm.at[pl.ds(base, B)], colv)
                pltpu.sync_copy(rows_hbm.at[pl.ds(base, B)], rowv)
                pltpu.sync_copy(vals_hbm.at[pl.ds(base, B)], valv)
                pltpu.async_copy(src.at[colv], rbuf, sem).wait()

                def group_body(g, _):
                    vv = valv[pl.ds(g * 16, 16)]
                    for t in range(16):
                        v = vv[t]
                        row = g * 16 + t
                        for kk in range(D_FEAT // 16):
                            sl = pl.ds(kk * 16, 16)
                            rbuf[row, sl] = rbuf[row, sl] * v
                    return 0
                lax.fori_loop(0, B // 16, group_body, 0)
                pltpu.sync_copy(rbuf, accum.at[rowv], add=True)
                return 0

            lax.fori_loop(0, NB, batch_body, 0)
            plsc.subcore_barrier()

            dst = out_hbm if layer == NUM_LAYERS - 1 else h_hbm
            flush_my_slice(dst)
            if layer < NUM_LAYERS - 1:
                zero_my_slice()
                plsc.subcore_barrier()

    return k(x, edge_row, edge_col, edge_vals)


def kernel(x, edge_row, edge_col, edge_vals):
    pad = E_PAD - N_EDGES
    edge_row = jnp.concatenate([edge_row, jnp.zeros((pad,), jnp.int32)])
    edge_col = jnp.concatenate([edge_col, jnp.zeros((pad,), jnp.int32)])
    edge_vals = jnp.concatenate([edge_vals, jnp.zeros((pad,), jnp.float32)])
    out, _ = _lightgcn_sc(x, edge_row, edge_col, edge_vals)
    return out


# speedup vs baseline: 3.6249x; 1.4489x over previous
import functools

import jax
import jax.numpy as jnp
from jax import lax
from jax.experimental import pallas as pl
from jax.experimental.pallas import tpu as pltpu
from jax.experimental.pallas import tpu_sc as plsc

N_NODES = 10000
N_EDGES = 320000
D_FEAT = 128
NUM_LAYERS = 3

NS = 16
B = 128
NB = 159
E_PAD = NS * NB * B
NBUF = 3
ROWS_PER_TILE = 624
TAIL_ROWS = N_NODES - NS * ROWS_PER_TILE


def _lightgcn_sc(x, edge_row, edge_col, edge_vals):
    mesh = plsc.VectorSubcoreMesh(
        core_axis_name="c", subcore_axis_name="s", num_cores=1)

    @functools.partial(
        pl.kernel,
        out_type=[
            jax.ShapeDtypeStruct((N_NODES, D_FEAT), jnp.float32),
            jax.ShapeDtypeStruct((N_NODES, D_FEAT), jnp.float32),
        ],
        mesh=mesh,
        scratch_types=[
            pltpu.VMEM_SHARED((N_NODES, D_FEAT), jnp.float32),
            [pltpu.VMEM((B,), jnp.int32)] * NBUF,
            [pltpu.VMEM((B,), jnp.int32)] * NBUF,
            [pltpu.VMEM((B,), jnp.float32)] * NBUF,
            [pltpu.VMEM((B, D_FEAT), jnp.float32)] * NBUF,
            [pltpu.SemaphoreType.DMA] * NBUF,
            [pltpu.SemaphoreType.DMA] * NBUF,
            [pltpu.SemaphoreType.DMA] * NBUF,
        ],
    )
    def k(x_hbm, rows_hbm, cols_hbm, vals_hbm, out_hbm, h_hbm,
          accum, colvs, rowvs, valvs, rbufs, isems, gsems, ssems):
        sid = lax.axis_index("s")
        zero16 = jnp.zeros((16,), jnp.float32)

        def fill_zeros(buf):
            def zrow(i, _):
                for kk in range(D_FEAT // 16):
                    buf[i, pl.ds(kk * 16, 16)] = zero16
                return 0
            lax.fori_loop(0, B, zrow, 0)

        def zero_my_slice(buf):
            for off, sz in [(0, B), (B, B), (2 * B, B), (3 * B, B),
                            (4 * B, 112)]:
                r0 = pl.multiple_of(sid * ROWS_PER_TILE + off, 8)
                pltpu.sync_copy(buf.at[pl.ds(0, sz)], accum.at[pl.ds(r0, sz)])

            @pl.when(sid == NS - 1)
            def _():
                pltpu.sync_copy(buf.at[pl.ds(0, TAIL_ROWS)],
                                accum.at[pl.ds(NS * ROWS_PER_TILE, TAIL_ROWS)])

        def flush_my_slice(dst_ref):
            for off, sz in [(0, B), (B, B), (2 * B, B), (3 * B, B),
                            (4 * B, 112)]:
                r0 = pl.multiple_of(sid * ROWS_PER_TILE + off, 8)
                pltpu.sync_copy(accum.at[pl.ds(r0, sz)],
                                dst_ref.at[pl.ds(r0, sz)])

            @pl.when(sid == NS - 1)
            def _():
                r0 = NS * ROWS_PER_TILE
                pltpu.sync_copy(accum.at[pl.ds(r0, TAIL_ROWS)],
                                dst_ref.at[pl.ds(r0, TAIL_ROWS)])

        def issue_idx(b, u):
            base = (sid * NB + b) * B
            pltpu.async_copy(cols_hbm.at[pl.ds(base, B)], colvs[u], isems[u])
            pltpu.async_copy(rows_hbm.at[pl.ds(base, B)], rowvs[u], isems[u])
            pltpu.async_copy(vals_hbm.at[pl.ds(base, B)], valvs[u], isems[u])

        def wait_idx(u):
            pltpu.make_async_copy(cols_hbm.at[pl.ds(0, B)], colvs[u],
                                  isems[u]).wait()
            pltpu.make_async_copy(rows_hbm.at[pl.ds(0, B)], rowvs[u],
                                  isems[u]).wait()
            pltpu.make_async_copy(vals_hbm.at[pl.ds(0, B)], valvs[u],
                                  isems[u]).wait()

        def issue_gather(src, u):
            pltpu.async_copy(src.at[colvs[u]], rbufs[u], gsems[u])

        def wait_gather(src, u):
            pltpu.make_async_copy(src.at[colvs[u]], rbufs[u], gsems[u]).wait()

        def issue_scatter(u):
            pltpu.async_copy(rbufs[u], accum.at[rowvs[u]], ssems[u], add=True)

        def wait_scatter(u):
            pltpu.make_async_copy(rbufs[u], accum.at[rowvs[u]],
                                  ssems[u]).wait()

        def scale(u):
            buf = rbufs[u]
            valv = valvs[u]

            def group_body(g, _):
                vv = valv[pl.ds(g * 16, 16)]
                for t in range(16):
                    v = vv[t]
                    row = g * 16 + t
                    for kk in range(D_FEAT // 16):
                        sl = pl.ds(kk * 16, 16)
                        buf[row, sl] = buf[row, sl] * v
                return 0
            lax.fori_loop(0, B // 16, group_body, 0)

        fill_zeros(rbufs[0])
        zero_my_slice(rbufs[0])
        plsc.subcore_barrier()

        for layer in range(NUM_LAYERS):
            src = x_hbm if layer == 0 else h_hbm

            for u in range(NBUF):
                issue_idx(u, u)
            for u in range(NBUF):
                wait_idx(u)
                issue_gather(src, u)

            def iter_body(i, _):
                for u in range(NBUF):
                    b = i * NBUF + u
                    prev = (u + NBUF - 1) % NBUF
                    more = jnp.logical_and(b >= 1, b + NBUF - 1 < NB)
                    wait_gather(src, u)

                    @pl.when(more)
                    def _():
                        wait_scatter(prev)
                        issue_idx(b + NBUF - 1, prev)

                    scale(u)
                    issue_scatter(u)

                    @pl.when(more)
                    def _():
                        wait_idx(prev)
                        issue_gather(src, prev)
                return 0

            lax.fori_loop(0, NB // NBUF, iter_body, 0)
            for u in range(NBUF):
                wait_scatter(u)
            plsc.subcore_barrier()

            dst = out_hbm if layer == NUM_LAYERS - 1 else h_hbm
            flush_my_slice(dst)
            if layer < NUM_LAYERS - 1:
                fill_zeros(rbufs[0])
                zero_my_slice(rbufs[0])
                plsc.subcore_barrier()

    return k(x, edge_row, edge_col, edge_vals)


def kernel(x, edge_row, edge_col, edge_vals):
    pad = E_PAD - N_EDGES
    edge_row = jnp.concatenate([edge_row, jnp.zeros((pad,), jnp.int32)])
    edge_col = jnp.concatenate([edge_col, jnp.zeros((pad,), jnp.int32)])
    edge_vals = jnp.concatenate([edge_vals, jnp.zeros((pad,), jnp.float32)])
    out, _ = _lightgcn_sc(x, edge_row, edge_col, edge_vals)
    return out
